# baseline (speedup 1.0000x reference)
import functools

import jax
import jax.numpy as jnp
from jax import lax
from jax.experimental import pallas as pl
from jax.experimental.pallas import tpu as pltpu
from jax.experimental.pallas import tpu_sc as plsc

N_NODES = 10000
D = 256
NT = 32
RPT = 320
DUMMY = RPT
E_PAD = 163840
EC = 2048
NCHUNK = E_PAD // EC
GB = 64
PAD_DST = NT * RPT


def _sc_segment_sum(x, src, dst):
    mesh = plsc.VectorSubcoreMesh(core_axis_name="c", subcore_axis_name="s")

    @functools.partial(
        pl.kernel,
        out_type=jax.ShapeDtypeStruct((N_NODES, D), jnp.float32),
        mesh=mesh,
        compiler_params=pltpu.CompilerParams(
            needs_layout_passes=False, use_tc_tiling_on_sc=False
        ),
        scratch_types=[
            pltpu.VMEM((2, EC), jnp.int32),
            pltpu.VMEM((2, EC), jnp.int32),
            pltpu.VMEM((EC + 160,), jnp.int32),
            pltpu.VMEM((EC + 160,), jnp.int32),
            pltpu.VMEM((2 * GB, D), jnp.float32),
            pltpu.VMEM((RPT + 8, D), jnp.float32),
            pltpu.SemaphoreType.DMA,
            pltpu.SemaphoreType.DMA,
        ],
    )
    def k(x_hbm, src_hbm, dst_hbm, out_hbm, sbuf, dbuf, msrc, mdst, rows, acc,
          csem, gsem):
        cid = lax.axis_index("c")
        sid = lax.axis_index("s")
        wid = sid * 2 + cid
        base = wid * RPT

        zf16 = jnp.zeros((16,), jnp.float32)
        iota = lax.iota(jnp.int32, 16)
        spread = wid * 16 + iota

        def zrow(r, carry):
            for c in range(D // 16):
                acc[r, pl.ds(c * 16, 16)] = zf16
            return carry
        lax.fori_loop(0, RPT + 8, zrow, 0)

        def zidx(i, carry):
            msrc[pl.ds(i * 16, 16)] = spread
            mdst[pl.ds(i * 16, 16)] = jnp.full((16,), DUMMY, jnp.int32)
            return carry
        lax.fori_loop(0, (EC + 160) // 16, zidx, 0)

        pltpu.async_copy(src_hbm.at[pl.ds(0, EC)], sbuf.at[0], csem)
        pltpu.async_copy(dst_hbm.at[pl.ds(0, EC)], dbuf.at[0], csem)

        def chunk_body(ch, carry):
            par = ch % 2
            nxt = (ch + 1) % NCHUNK

            pltpu.make_async_copy(src_hbm.at[pl.ds(0, EC)], sbuf.at[par], csem).wait()
            pltpu.make_async_copy(dst_hbm.at[pl.ds(0, EC)], dbuf.at[par], csem).wait()
            pltpu.async_copy(src_hbm.at[pl.ds(nxt * EC, EC)], sbuf.at[1 - par], csem)
            pltpu.async_copy(dst_hbm.at[pl.ds(nxt * EC, EC)], dbuf.at[1 - par], csem)

            def scan_body(g, cntv):
                d0 = dbuf[par, pl.ds(g * 32, 16)]
                s0 = sbuf[par, pl.ds(g * 32, 16)]
                m0 = (d0 >= base) & (d0 < base + RPT)
                pos0 = cntv + plsc.cumsum(m0.astype(jnp.int32)) - 1
                plsc.store_scatter(msrc, [pos0], s0, mask=m0)
                plsc.store_scatter(mdst, [pos0], d0 - base, mask=m0)
                cntv2 = cntv + plsc.all_reduce_population_count(m0)

                d1 = dbuf[par, pl.ds(g * 32 + 16, 16)]
                s1 = sbuf[par, pl.ds(g * 32 + 16, 16)]
                m1 = (d1 >= base) & (d1 < base + RPT)
                pos1 = cntv2 + plsc.cumsum(m1.astype(jnp.int32)) - 1
                plsc.store_scatter(msrc, [pos1], s1, mask=m1)
                plsc.store_scatter(mdst, [pos1], d1 - base, mask=m1)
                return cntv2 + plsc.all_reduce_population_count(m1)

            cntv = lax.fori_loop(0, EC // 32, scan_body,
                                 jnp.zeros((16,), jnp.int32))
            cnt = cntv[0]

            msrc[pl.ds(cnt, 16)] = spread
            mdst[pl.ds(cnt, 16)] = jnp.full((16,), DUMMY, jnp.int32)
            cntp = (cnt + 15) & ~15

            nsb = (cntp + GB - 1) // GB

            pltpu.async_copy(x_hbm.at[msrc.at[pl.ds(0, GB)]],
                             rows.at[pl.ds(0, GB)], gsem)

            def sub_body(sbt, carry2):
                h = sbt % 2

                @pl.when(sbt + 1 < nsb)
                def _():
                    pltpu.async_copy(
                        x_hbm.at[msrc.at[pl.ds((sbt + 1) * GB, GB)]],
                        rows.at[pl.ds((1 - h) * GB, GB)], gsem)

                pltpu.make_async_copy(x_hbm.at[msrc.at[pl.ds(0, GB)]],
                                      rows.at[pl.ds(0, GB)], gsem).wait()

                def group_body(g, carry3):
                    ld = mdst[pl.ds(sbt * GB + g * 16, 16)]
                    rbase = h * GB + g * 16
                    for l0 in range(0, 16, 4):
                        r0 = ld[l0]
                        r1 = ld[l0 + 1]
                        r2 = ld[l0 + 2]
                        r3 = ld[l0 + 3]
                        for c in range(D // 16):
                            cs = pl.ds(c * 16, 16)
                            plsc.addupdate(acc.at[r0, cs], rows[rbase + l0, cs])
                            plsc.addupdate(acc.at[r1, cs], rows[rbase + l0 + 1, cs])
                            plsc.addupdate(acc.at[r2, cs], rows[rbase + l0 + 2, cs])
                            plsc.addupdate(acc.at[r3, cs], rows[rbase + l0 + 3, cs])
                    return carry3

                gmax = lax.min((cntp - sbt * GB + 15) // 16, GB // 16)
                lax.fori_loop(0, gmax, group_body, 0)
                return carry2

            lax.fori_loop(0, nsb, sub_body, 0)
            return carry

        lax.fori_loop(0, NCHUNK, chunk_body, 0)

        pltpu.make_async_copy(src_hbm.at[pl.ds(0, EC)], sbuf.at[0], csem).wait()
        pltpu.make_async_copy(dst_hbm.at[pl.ds(0, EC)], dbuf.at[0], csem).wait()

        @pl.when(wid != NT - 1)
        def _():
            pltpu.sync_copy(acc.at[pl.ds(0, RPT)], out_hbm.at[pl.ds(base, RPT)])

        @pl.when(wid == NT - 1)
        def _():
            last = N_NODES - (NT - 1) * RPT
            pltpu.sync_copy(acc.at[pl.ds(0, last)], out_hbm.at[pl.ds(base, last)])

    return k(x, src, dst)


def _mm_body(a_ref, w_ref, b_ref, o_ref):
    out = jnp.dot(a_ref[...], w_ref[...], preferred_element_type=jnp.float32)
    o_ref[...] = jnp.maximum(out + b_ref[...], 0.0)


def _tc_linear_relu(agg, W, b):
    blk = 1000
    return pl.pallas_call(
        _mm_body,
        grid=(N_NODES // blk,),
        in_specs=[
            pl.BlockSpec((blk, D), lambda i: (i, 0)),
            pl.BlockSpec((D, D), lambda i: (0, 0)),
            pl.BlockSpec((1, D), lambda i: (0, 0)),
        ],
        out_specs=pl.BlockSpec((blk, D), lambda i: (i, 0)),
        out_shape=jax.ShapeDtypeStruct((N_NODES, D), jnp.float32),
    )(agg, W, b.reshape(1, D))


def kernel(x, edge_index, W, b):
    src = edge_index[0].astype(jnp.int32)
    dst = edge_index[1].astype(jnp.int32)
    pad = E_PAD - src.shape[0]
    src = jnp.concatenate([src, jnp.zeros((pad,), jnp.int32)])
    dst = jnp.concatenate([dst, jnp.full((pad,), PAD_DST, jnp.int32)])
    agg = _sc_segment_sum(x, src, dst)
    return _tc_linear_relu(agg, W, b)


# speedup vs baseline: 1.6792x; 1.6792x over previous
import functools

import jax
import jax.numpy as jnp
from jax import lax
from jax.experimental import pallas as pl
from jax.experimental.pallas import tpu as pltpu
from jax.experimental.pallas import tpu_sc as plsc

N_NODES = 10000
D = 256
NT = 32
RPT = 320
DUMMY = RPT
E_PAD = 163840
EC = 2048
NCHUNK = E_PAD // EC
GB = 48
PAD_DST = NT * RPT


def _sc_segment_sum(x, src, dst):
    mesh = plsc.VectorSubcoreMesh(core_axis_name="c", subcore_axis_name="s")

    @functools.partial(
        pl.kernel,
        out_type=jax.ShapeDtypeStruct((N_NODES, D), jnp.float32),
        mesh=mesh,
        compiler_params=pltpu.CompilerParams(
            needs_layout_passes=False, use_tc_tiling_on_sc=False
        ),
        scratch_types=[
            pltpu.VMEM((2, EC), jnp.int32),
            pltpu.VMEM((2, EC), jnp.int32),
            pltpu.VMEM((16 * 128,), jnp.int32),
            pltpu.VMEM((2 * (EC + 64),), jnp.int32),
            pltpu.VMEM((2 * (EC + 64),), jnp.int32),
            pltpu.VMEM((2 * GB, D), jnp.float32),
            pltpu.VMEM((RPT + 8, D), jnp.float32),
            pltpu.SemaphoreType.DMA,
            pltpu.SemaphoreType.DMA,
        ],
    )
    def k(x_hbm, src_hbm, dst_hbm, out_hbm, sbuf, dbuf, regs, msrc, mdst, rows,
          acc, csem, gsem):
        cid = lax.axis_index("c")
        sid = lax.axis_index("s")
        wid = sid * 2 + cid
        base = wid * RPT

        zf16 = jnp.zeros((16,), jnp.float32)
        iota = lax.iota(jnp.int32, 16)
        spread = wid * 16 + iota

        def zrow(r, carry):
            for c in range(D // 16):
                acc[r, pl.ds(c * 16, 16)] = zf16
            return carry
        lax.fori_loop(0, RPT + 8, zrow, 0)

        def zidx(i, carry):
            msrc[pl.ds(i * 16, 16)] = spread
            mdst[pl.ds(i * 16, 16)] = jnp.full((16,), DUMMY, jnp.int32)
            return carry
        lax.fori_loop(0, 2 * (EC + 64) // 16, zidx, 0)

        pltpu.async_copy(src_hbm.at[pl.ds(0, EC)], sbuf.at[0], csem)
        pltpu.async_copy(dst_hbm.at[pl.ds(0, EC)], dbuf.at[0], csem)

        LW = EC + 64

        def chunk_body(ch, pcntp):
            par = ch % 2
            q = 1 - par
            nxt = (ch + 1) % NCHUNK

            pltpu.make_async_copy(src_hbm.at[pl.ds(0, EC)], sbuf.at[par], csem).wait()
            pltpu.make_async_copy(dst_hbm.at[pl.ds(0, EC)], dbuf.at[par], csem).wait()
            pltpu.async_copy(src_hbm.at[pl.ds(nxt * EC, EC)], sbuf.at[1 - par], csem)
            pltpu.async_copy(dst_hbm.at[pl.ds(nxt * EC, EC)], dbuf.at[1 - par], csem)

            pnsb = (pcntp + GB - 1) // GB

            @pl.when(pnsb > 0)
            def _():
                pltpu.async_copy(x_hbm.at[msrc.at[pl.ds(q * LW, GB)]],
                                 rows.at[pl.ds(0, GB)], gsem)

            slot0 = iota * 128

            def scan_body(g, counts):
                for u in range(4):
                    d = dbuf[par, pl.ds(g * 64 + u * 16, 16)]
                    s = sbuf[par, pl.ds(g * 64 + u * 16, 16)]
                    m = (d >= base) & (d < base + RPT)
                    packed = lax.shift_left(s, 9) | (d - base)
                    plsc.store_scatter(regs, [slot0 + counts], packed, mask=m)
                    counts = counts + m.astype(jnp.int32)
                return counts

            counts = lax.fori_loop(0, EC // 64, scan_body,
                                   jnp.zeros((16,), jnp.int32))

            def sub_body(sbt, carry2):
                h = sbt % 2

                @pl.when(sbt + 1 < pnsb)
                def _():
                    pltpu.async_copy(
                        x_hbm.at[msrc.at[pl.ds(q * LW + (sbt + 1) * GB, GB)]],
                        rows.at[pl.ds((1 - h) * GB, GB)], gsem)

                pltpu.make_async_copy(x_hbm.at[msrc.at[pl.ds(0, GB)]],
                                      rows.at[pl.ds(0, GB)], gsem).wait()

                def group_body(g, carry3):
                    ld = mdst[pl.ds(q * LW + sbt * GB + g * 16, 16)]
                    rbase = h * GB + g * 16
                    for l in range(16):
                        r = ld[l]
                        vals = [rows[rbase + l, pl.ds(c * 16, 16)]
                                for c in range(D // 16)]
                        for c in range(D // 16):
                            plsc.addupdate(acc.at[r, pl.ds(c * 16, 16)], vals[c])
                    return carry3

                gmax = lax.min((pcntp - sbt * GB + 15) // 16, GB // 16)
                lax.fori_loop(0, gmax, group_body, 0)
                return carry2

            lax.fori_loop(0, pnsb, sub_body, 0)

            cnt = 0
            for r in range(16):
                c_r = counts[r]
                p_r = cnt

                def merge_body(j, carry, _r=r, _p=p_r):
                    v = regs[pl.ds(_r * 128 + j * 16, 16)]
                    msrc[pl.ds(par * LW + _p + j * 16, 16)] = \
                        lax.shift_right_logical(v, 9)
                    mdst[pl.ds(par * LW + _p + j * 16, 16)] = v & 511
                    return carry

                nj = lax.max(1, (c_r + 15) // 16)
                lax.fori_loop(0, nj, merge_body, 0)
                cnt = cnt + c_r

            msrc[pl.ds(par * LW + cnt, 16)] = spread
            mdst[pl.ds(par * LW + cnt, 16)] = jnp.full((16,), DUMMY, jnp.int32)
            return (cnt + 15) & ~15

        lax.fori_loop(0, NCHUNK + 1, chunk_body, 0)

        pltpu.make_async_copy(src_hbm.at[pl.ds(0, EC)], sbuf.at[0], csem).wait()
        pltpu.make_async_copy(dst_hbm.at[pl.ds(0, EC)], dbuf.at[0], csem).wait()

        @pl.when(wid != NT - 1)
        def _():
            pltpu.sync_copy(acc.at[pl.ds(0, RPT)], out_hbm.at[pl.ds(base, RPT)])

        @pl.when(wid == NT - 1)
        def _():
            last = N_NODES - (NT - 1) * RPT
            pltpu.sync_copy(acc.at[pl.ds(0, last)], out_hbm.at[pl.ds(base, last)])

    return k(x, src, dst)


def _mm_body(a_ref, w_ref, b_ref, o_ref):
    out = jnp.dot(a_ref[...], w_ref[...], preferred_element_type=jnp.float32)
    o_ref[...] = jnp.maximum(out + b_ref[...], 0.0)


def _tc_linear_relu(agg, W, b):
    blk = 1000
    return pl.pallas_call(
        _mm_body,
        grid=(N_NODES // blk,),
        in_specs=[
            pl.BlockSpec((blk, D), lambda i: (i, 0)),
            pl.BlockSpec((D, D), lambda i: (0, 0)),
            pl.BlockSpec((1, D), lambda i: (0, 0)),
        ],
        out_specs=pl.BlockSpec((blk, D), lambda i: (i, 0)),
        out_shape=jax.ShapeDtypeStruct((N_NODES, D), jnp.float32),
    )(agg, W, b.reshape(1, D))


def kernel(x, edge_index, W, b):
    src = edge_index[0].astype(jnp.int32)
    dst = edge_index[1].astype(jnp.int32)
    pad = E_PAD - src.shape[0]
    src = jnp.concatenate([src, jnp.zeros((pad,), jnp.int32)])
    dst = jnp.concatenate([dst, jnp.full((pad,), PAD_DST, jnp.int32)])
    agg = _sc_segment_sum(x, src, dst)
    return _tc_linear_relu(agg, W, b)


# speedup vs baseline: 1.9035x; 1.1336x over previous
import functools

import jax
import jax.numpy as jnp
from jax import lax
from jax.experimental import pallas as pl
from jax.experimental.pallas import tpu as pltpu
from jax.experimental.pallas import tpu_sc as plsc

N_NODES = 10000
D = 256
NT = 32
RPT = 320
DUMMY = RPT
E_PAD = 163840
EC = 2560
NCHUNK = E_PAD // EC
GB = 48
PAD_DST = NT * RPT


def _sc_segment_sum(x, src, dst):
    mesh = plsc.VectorSubcoreMesh(core_axis_name="c", subcore_axis_name="s")

    @functools.partial(
        pl.kernel,
        out_type=jax.ShapeDtypeStruct((N_NODES, D), jnp.float32),
        mesh=mesh,
        compiler_params=pltpu.CompilerParams(
            needs_layout_passes=False, use_tc_tiling_on_sc=False
        ),
        scratch_types=[
            pltpu.VMEM((2, EC), jnp.int32),
            pltpu.VMEM((2, EC), jnp.int32),
            pltpu.VMEM((EC,), jnp.int32),
            pltpu.VMEM((EC + 64,), jnp.int32),
            pltpu.VMEM((EC + 64,), jnp.int32),
            pltpu.VMEM((2 * GB, D), jnp.float32),
            pltpu.VMEM((RPT + 8, D), jnp.float32),
            pltpu.SemaphoreType.DMA,
            pltpu.SemaphoreType.DMA,
        ],
    )
    def k(x_hbm, src_hbm, dst_hbm, out_hbm, sbuf, dbuf, regs, msrc, mdst, rows,
          acc, csem, gsem):
        cid = lax.axis_index("c")
        sid = lax.axis_index("s")
        wid = sid * 2 + cid
        base = wid * RPT

        zf16 = jnp.zeros((16,), jnp.float32)
        iota = lax.iota(jnp.int32, 16)
        spread = wid * 16 + iota

        def zrow(r, carry):
            for c in range(D // 16):
                acc[r, pl.ds(c * 16, 16)] = zf16
            return carry
        lax.fori_loop(0, RPT + 8, zrow, 0)

        def zidx(i, carry):
            msrc[pl.ds(i * 16, 16)] = spread
            mdst[pl.ds(i * 16, 16)] = jnp.full((16,), DUMMY, jnp.int32)
            return carry
        lax.fori_loop(0, (EC + 64) // 16, zidx, 0)

        pltpu.async_copy(src_hbm.at[pl.ds(0, EC)], sbuf.at[0], csem)
        pltpu.async_copy(dst_hbm.at[pl.ds(0, EC)], dbuf.at[0], csem)

        def chunk_body(ch, carry):
            par = ch % 2
            nxt = (ch + 1) % NCHUNK

            pltpu.make_async_copy(src_hbm.at[pl.ds(0, EC)], sbuf.at[par], csem).wait()
            pltpu.make_async_copy(dst_hbm.at[pl.ds(0, EC)], dbuf.at[par], csem).wait()
            pltpu.async_copy(src_hbm.at[pl.ds(nxt * EC, EC)], sbuf.at[1 - par], csem)
            pltpu.async_copy(dst_hbm.at[pl.ds(nxt * EC, EC)], dbuf.at[1 - par], csem)

            slot0 = iota * (EC // 16)

            def scan_body(g, counts):
                for u in range(4):
                    d = dbuf[par, pl.ds(g * 64 + u * 16, 16)]
                    s = sbuf[par, pl.ds(g * 64 + u * 16, 16)]
                    m = (d >= base) & (d < base + RPT)
                    packed = lax.shift_left(s, 9) | (d - base)
                    plsc.store_scatter(regs, [slot0 + counts], packed, mask=m)
                    counts = counts + m.astype(jnp.int32)
                return counts

            counts = lax.fori_loop(0, EC // 64, scan_body,
                                   jnp.zeros((16,), jnp.int32))

            cnt = 0
            for r in range(16):
                c_r = counts[r]
                p_r = cnt

                def merge_body(j, carry, _r=r, _p=p_r):
                    v = regs[pl.ds(_r * (EC // 16) + j * 16, 16)]
                    msrc[pl.ds(_p + j * 16, 16)] = lax.shift_right_logical(v, 9)
                    mdst[pl.ds(_p + j * 16, 16)] = v & 511
                    return carry

                nj = lax.max(1, (c_r + 15) // 16)
                lax.fori_loop(0, nj, merge_body, 0)
                cnt = cnt + c_r

            msrc[pl.ds(cnt, 16)] = spread
            mdst[pl.ds(cnt, 16)] = jnp.full((16,), DUMMY, jnp.int32)
            cntp = (cnt + 15) & ~15

            nsb = (cntp + GB - 1) // GB

            pltpu.async_copy(x_hbm.at[msrc.at[pl.ds(0, GB)]],
                             rows.at[pl.ds(0, GB)], gsem)

            def sub_body(sbt, carry2):
                h = sbt % 2

                @pl.when(sbt + 1 < nsb)
                def _():
                    pltpu.async_copy(
                        x_hbm.at[msrc.at[pl.ds((sbt + 1) * GB, GB)]],
                        rows.at[pl.ds((1 - h) * GB, GB)], gsem)

                pltpu.make_async_copy(x_hbm.at[msrc.at[pl.ds(0, GB)]],
                                      rows.at[pl.ds(0, GB)], gsem).wait()

                def group_body(g, carry3):
                    ld = mdst[pl.ds(sbt * GB + g * 16, 16)]
                    rbase = h * GB + g * 16
                    for l in range(16):
                        r = ld[l]
                        vals = [rows[rbase + l, pl.ds(c * 16, 16)]
                                for c in range(D // 16)]
                        for c in range(D // 16):
                            plsc.addupdate(acc.at[r, pl.ds(c * 16, 16)], vals[c])
                    return carry3

                gmax = lax.min((cntp - sbt * GB + 15) // 16, GB // 16)
                lax.fori_loop(0, gmax, group_body, 0)
                return carry2

            lax.fori_loop(0, nsb, sub_body, 0)
            return carry

        lax.fori_loop(0, NCHUNK, chunk_body, 0)

        pltpu.make_async_copy(src_hbm.at[pl.ds(0, EC)], sbuf.at[0], csem).wait()
        pltpu.make_async_copy(dst_hbm.at[pl.ds(0, EC)], dbuf.at[0], csem).wait()

        @pl.when(wid != NT - 1)
        def _():
            pltpu.sync_copy(acc.at[pl.ds(0, RPT)], out_hbm.at[pl.ds(base, RPT)])

        @pl.when(wid == NT - 1)
        def _():
            last = N_NODES - (NT - 1) * RPT
            pltpu.sync_copy(acc.at[pl.ds(0, last)], out_hbm.at[pl.ds(base, last)])

    return k(x, src, dst)


def _mm_body(a_ref, w_ref, b_ref, o_ref):
    out = jnp.dot(a_ref[...], w_ref[...], preferred_element_type=jnp.float32)
    o_ref[...] = jnp.maximum(out + b_ref[...], 0.0)


def _tc_linear_relu(agg, W, b):
    blk = 1000
    return pl.pallas_call(
        _mm_body,
        grid=(N_NODES // blk,),
        in_specs=[
            pl.BlockSpec((blk, D), lambda i: (i, 0)),
            pl.BlockSpec((D, D), lambda i: (0, 0)),
            pl.BlockSpec((1, D), lambda i: (0, 0)),
        ],
        out_specs=pl.BlockSpec((blk, D), lambda i: (i, 0)),
        out_shape=jax.ShapeDtypeStruct((N_NODES, D), jnp.float32),
    )(agg, W, b.reshape(1, D))


def kernel(x, edge_index, W, b):
    src = edge_index[0].astype(jnp.int32)
    dst = edge_index[1].astype(jnp.int32)
    pad = E_PAD - src.shape[0]
    src = jnp.concatenate([src, jnp.zeros((pad,), jnp.int32)])
    dst = jnp.concatenate([dst, jnp.full((pad,), PAD_DST, jnp.int32)])
    agg = _sc_segment_sum(x, src, dst)
    return _tc_linear_relu(agg, W, b)
